# Initial kernel scaffold; baseline (speedup 1.0000x reference)
import jax
import jax.numpy as jnp
from jax.experimental import pallas as pl


def kernel(hid, segment_ids, W_ih, W_hh, b_ih, b_hh, W1, b1, W2, b2):
    raise NotImplementedError("write your pallas kernel here")


# speedup vs baseline: 30.9160x; 30.9160x over previous
import functools

import jax
import jax.numpy as jnp
from jax.experimental import pallas as pl
from jax.experimental.pallas import tpu as pltpu

H = 128
B = 256
N = 100000
OUT = 10
R = 2000
NBLK = N // R


def _pool_mlp_kernel(hid_ref, seg_ref, bih_ref, bhh_ref, w1_ref, b1_ref,
                     w2_ref, b2_ref, out_ref, d_acc, s_acc):
    i = pl.program_id(0)

    @pl.when(i == 0)
    def _init():
        d_acc[...] = jnp.zeros_like(d_acc)
        s_acc[...] = jnp.zeros_like(s_acc)

    gates = bih_ref[...] + bhh_ref[...]
    i_g = jax.nn.sigmoid(gates[0:1, :])
    g_g = jnp.tanh(gates[2:3, :])
    o_g = jax.nn.sigmoid(gates[3:4, :])
    c = i_g * g_g
    q = o_g * jnp.tanh(c)

    hid_blk = hid_ref[...]
    seg = seg_ref[...].reshape(1, R)

    e = jax.lax.dot_general(q, hid_blk, (((1,), (1,)), ((), ())),
                            preferred_element_type=jnp.float32)
    w = jnp.exp(e)

    seg_iota = jax.lax.broadcasted_iota(jnp.int32, (B, 1), 0)
    mask = seg == seg_iota
    w_mat = jnp.where(mask, w, 0.0)

    d_acc[...] += jnp.sum(w_mat, axis=1, keepdims=True)
    s_acc[...] += jax.lax.dot_general(
        w_mat, hid_blk, (((1,), (0,)), ((), ())),
        preferred_element_type=jnp.float32)

    @pl.when(i == NBLK - 1)
    def _epilogue():
        d = d_acc[...]
        readout = jnp.where(d > 0.0, s_acc[...] / d, 0.0)
        w1 = w1_ref[...]
        q_part = jax.lax.dot_general(q, w1[:, :H], (((1,), (1,)), ((), ())),
                                     preferred_element_type=jnp.float32)
        r_part = jax.lax.dot_general(readout, w1[:, H:],
                                     (((1,), (1,)), ((), ())),
                                     preferred_element_type=jnp.float32)
        pre1 = q_part + r_part + b1_ref[...]
        x1 = jnp.where(pre1 > 0.0, pre1, jnp.exp(pre1) - 1.0)
        x2 = jax.lax.dot_general(x1, w2_ref[...], (((1,), (1,)), ((), ())),
                                 preferred_element_type=jnp.float32)
        pre2 = x2 + b2_ref[...]
        out_ref[...] = jnp.where(pre2 > 0.0, pre2, jnp.exp(pre2) - 1.0)


@functools.partial(jax.jit, static_argnames=())
def _run(hid, seg3d, bih2, bhh2, W1, b1r, W2p, b2p):
    out16 = pl.pallas_call(
        _pool_mlp_kernel,
        grid=(NBLK,),
        in_specs=[
            pl.BlockSpec((R, H), lambda i: (i, 0)),
            pl.BlockSpec((1, 1, R), lambda i: (i, 0, 0)),
            pl.BlockSpec((4, H), lambda i: (0, 0)),
            pl.BlockSpec((4, H), lambda i: (0, 0)),
            pl.BlockSpec((H, 2 * H), lambda i: (0, 0)),
            pl.BlockSpec((1, H), lambda i: (0, 0)),
            pl.BlockSpec((16, H), lambda i: (0, 0)),
            pl.BlockSpec((1, 16), lambda i: (0, 0)),
        ],
        out_specs=pl.BlockSpec((B, 16), lambda i: (0, 0)),
        out_shape=jax.ShapeDtypeStruct((B, 16), jnp.float32),
        scratch_shapes=[
            pltpu.VMEM((B, 1), jnp.float32),
            pltpu.VMEM((B, H), jnp.float32),
        ],
    )(hid, seg3d, bih2, bhh2, W1, b1r, W2p, b2p)
    return out16[:, :OUT]


def kernel(hid, segment_ids, W_ih, W_hh, b_ih, b_hh, W1, b1, W2, b2):
    seg3d = segment_ids.astype(jnp.int32).reshape(NBLK, 1, R)
    bih2 = b_ih.reshape(4, H)
    bhh2 = b_hh.reshape(4, H)
    b1r = b1.reshape(1, H)
    W2p = jnp.zeros((16, H), jnp.float32).at[:OUT].set(W2)
    b2p = jnp.zeros((1, 16), jnp.float32).at[0, :OUT].set(b2)
    return _run(hid, seg3d, bih2, bhh2, W1, b1r, W2p, b2p)
